# baseline (speedup 1.0000x reference)
import functools

import jax
import jax.numpy as jnp
from jax.experimental import pallas as pl

_TOP_K = 32


def _xt_kernel(x_ref, w_ref, out_ref):
    xt = jnp.dot(x_ref[0], w_ref[...], preferred_element_type=jnp.float32)
    norm = jnp.sqrt(jnp.sum(xt * xt, axis=1, keepdims=True))
    out_ref[0] = xt / jnp.maximum(norm, 1e-12)


def _scores_kernel(xt_blk_ref, xt_all_ref, out_ref):
    xb = xt_blk_ref[0]
    xa = xt_all_ref[0]
    s = jax.lax.dot_general(
        xb, xa, (((1,), (1,)), ((), ())),
        preferred_element_type=jnp.float32)
    s = jnp.maximum(s, 0.0)

    bm, n = s.shape
    ones_bf = jnp.ones((n, 8), jnp.bfloat16)
    lo = jnp.zeros((bm, 1), jnp.int32)
    hi = jnp.full((bm, 1), 0x40000000, jnp.int32)

    def body(_, lohi):
        lo, hi = lohi
        mid = (lo + hi) >> 1
        t = jax.lax.bitcast_convert_type(mid, jnp.float32)
        mask_bf = (s >= t).astype(jnp.bfloat16)
        cnt = jax.lax.dot_general(
            mask_bf, ones_bf, (((1,), (0,)), ((), ())),
            preferred_element_type=jnp.float32)[:, :1]
        ge = cnt >= float(_TOP_K)
        return jnp.where(ge, mid, lo), jnp.where(ge, hi, mid)

    lo, hi = jax.lax.fori_loop(0, 30, body, (lo, hi))
    thresh = jax.lax.bitcast_convert_type(lo, jnp.float32)

    e = jnp.where(s >= thresh, jnp.exp(s), 1.0)
    out_ref[0] = e / jnp.sum(e, axis=1, keepdims=True)


@functools.partial(jax.jit, static_argnames=())
def kernel(x, W):
    B, N, D = x.shape
    bm1 = 256
    xt = pl.pallas_call(
        _xt_kernel,
        grid=(B, N // bm1),
        in_specs=[
            pl.BlockSpec((1, bm1, D), lambda b, i: (b, i, 0)),
            pl.BlockSpec((D, D), lambda b, i: (0, 0)),
        ],
        out_specs=pl.BlockSpec((1, bm1, D), lambda b, i: (b, i, 0)),
        out_shape=jax.ShapeDtypeStruct((B, N, D), jnp.float32),
    )(x, W)

    bm2 = 256
    out = pl.pallas_call(
        _scores_kernel,
        grid=(B, N // bm2),
        in_specs=[
            pl.BlockSpec((1, bm2, D), lambda b, i: (b, i, 0)),
            pl.BlockSpec((1, N, D), lambda b, i: (b, 0, 0)),
        ],
        out_specs=pl.BlockSpec((1, bm2, N), lambda b, i: (b, i, 0)),
        out_shape=jax.ShapeDtypeStruct((B, N, N), jnp.float32),
    )(xt, xt)
    return out


# speedup vs baseline: 1.3655x; 1.3655x over previous
import functools

import jax
import jax.numpy as jnp
from jax.experimental import pallas as pl

_TOP_K = 32


def _xt_kernel(x_ref, w_ref, out_ref):
    xt = jnp.dot(x_ref[0], w_ref[...], preferred_element_type=jnp.float32)
    norm = jnp.sqrt(jnp.sum(xt * xt, axis=1, keepdims=True))
    out_ref[0] = xt / jnp.maximum(norm, 1e-12)


def _scores_kernel(xt_all_ref, xt_blk_ref, out_ref):
    xa = xt_all_ref[0]
    xb = xt_blk_ref[0]
    st = jax.lax.dot_general(
        xa, xb, (((1,), (1,)), ((), ())),
        preferred_element_type=jnp.float32)
    st = jnp.maximum(st, 0.0)

    bm = st.shape[1]
    lo = jnp.zeros((1, bm), jnp.int32)
    hi = jnp.full((1, bm), 0x40000000, jnp.int32)

    def body(_, lohi):
        lo, hi = lohi
        mid = (lo + hi) >> 1
        t = jax.lax.bitcast_convert_type(mid, jnp.float32)
        cnt = jnp.sum((st >= t).astype(jnp.float32), axis=0, keepdims=True)
        ge = cnt >= float(_TOP_K)
        return jnp.where(ge, mid, lo), jnp.where(ge, hi, mid)

    lo, hi = jax.lax.fori_loop(0, 30, body, (lo, hi))
    thresh = jax.lax.bitcast_convert_type(lo, jnp.float32)

    e = jnp.where(st >= thresh, jnp.exp(st), 1.0)
    out_ref[0] = e / jnp.sum(e, axis=0, keepdims=True)


@functools.partial(jax.jit, static_argnames=())
def kernel(x, W):
    B, N, D = x.shape
    bm1 = 256
    xt = pl.pallas_call(
        _xt_kernel,
        grid=(B, N // bm1),
        in_specs=[
            pl.BlockSpec((1, bm1, D), lambda b, i: (b, i, 0)),
            pl.BlockSpec((D, D), lambda b, i: (0, 0)),
        ],
        out_specs=pl.BlockSpec((1, bm1, D), lambda b, i: (b, i, 0)),
        out_shape=jax.ShapeDtypeStruct((B, N, D), jnp.float32),
    )(x, W)

    bm2 = 256
    out_t = pl.pallas_call(
        _scores_kernel,
        grid=(B, N // bm2),
        in_specs=[
            pl.BlockSpec((1, N, D), lambda b, i: (b, 0, 0)),
            pl.BlockSpec((1, bm2, D), lambda b, i: (b, i, 0)),
        ],
        out_specs=pl.BlockSpec((1, N, bm2), lambda b, i: (b, 0, i)),
        out_shape=jax.ShapeDtypeStruct((B, N, N), jnp.float32),
    )(xt, xt)
    return jnp.swapaxes(out_t, 1, 2)


# speedup vs baseline: 1.4434x; 1.0571x over previous
import functools

import jax
import jax.numpy as jnp
from jax.experimental import pallas as pl

_TOP_K = 32


def _xt_kernel(x_ref, w_ref, out_ref):
    xt = jnp.dot(x_ref[0], w_ref[...], preferred_element_type=jnp.float32)
    norm = jnp.sqrt(jnp.sum(xt * xt, axis=1, keepdims=True))
    out_ref[0] = xt / jnp.maximum(norm, 1e-12)


def _colsum_i16(x):
    n = x.shape[0]
    while n > 16:
        n //= 2
        x = x[:n] + x[n:]
    return jnp.sum(x.astype(jnp.int32), axis=0, keepdims=True)


def _scores_kernel(xt_all_ref, xt_blk_ref, out_ref):
    xa = xt_all_ref[0]
    xb = xt_blk_ref[0]
    st = jax.lax.dot_general(
        xa, xb, (((1,), (1,)), ((), ())),
        preferred_element_type=jnp.float32)
    st = jnp.maximum(st, 0.0)

    bm = st.shape[1]
    bits = jax.lax.bitcast_convert_type(st, jnp.int32)
    top16 = (bits >> 16).astype(jnp.int16)
    low16 = ((bits & 0xFFFF) - 32768).astype(jnp.int16)

    lo_t = jnp.zeros((1, bm), jnp.int32)
    hi_t = jnp.full((1, bm), 0x4000, jnp.int32)

    def body_top(_, lohi):
        lo, hi = lohi
        mid = (lo + hi) >> 1
        t = mid.astype(jnp.int16)
        cnt = _colsum_i16((top16 >= t).astype(jnp.int16))
        ge = cnt >= _TOP_K
        return jnp.where(ge, mid, lo), jnp.where(ge, hi, mid)

    lo_t, hi_t = jax.lax.fori_loop(0, 14, body_top, (lo_t, hi_t))

    t_top = lo_t.astype(jnp.int16)
    in_bucket = top16 == t_top
    cnt_above = _colsum_i16((top16 > t_top).astype(jnp.int16))
    need = _TOP_K - cnt_above

    lo_l = jnp.full((1, bm), -32768, jnp.int32)
    hi_l = jnp.full((1, bm), 32768, jnp.int32)

    def body_low(_, lohi):
        lo, hi = lohi
        mid = (lo + hi) >> 1
        t = mid.astype(jnp.int16)
        cnt = _colsum_i16((in_bucket & (low16 >= t)).astype(jnp.int16))
        ge = cnt >= need
        return jnp.where(ge, mid, lo), jnp.where(ge, hi, mid)

    lo_l, hi_l = jax.lax.fori_loop(0, 16, body_low, (lo_l, hi_l))

    thresh_bits = (lo_t << 16) | (lo_l + 32768)
    thresh = jax.lax.bitcast_convert_type(thresh_bits, jnp.float32)

    e = jnp.where(st >= thresh, jnp.exp(st), 1.0)
    out_ref[0] = e / jnp.sum(e, axis=0, keepdims=True)


@functools.partial(jax.jit, static_argnames=())
def kernel(x, W):
    B, N, D = x.shape
    bm1 = 256
    xt = pl.pallas_call(
        _xt_kernel,
        grid=(B, N // bm1),
        in_specs=[
            pl.BlockSpec((1, bm1, D), lambda b, i: (b, i, 0)),
            pl.BlockSpec((D, D), lambda b, i: (0, 0)),
        ],
        out_specs=pl.BlockSpec((1, bm1, D), lambda b, i: (b, i, 0)),
        out_shape=jax.ShapeDtypeStruct((B, N, D), jnp.float32),
    )(x, W)

    bm2 = 256
    out_t = pl.pallas_call(
        _scores_kernel,
        grid=(B, N // bm2),
        in_specs=[
            pl.BlockSpec((1, N, D), lambda b, i: (b, 0, 0)),
            pl.BlockSpec((1, bm2, D), lambda b, i: (b, i, 0)),
        ],
        out_specs=pl.BlockSpec((1, N, bm2), lambda b, i: (b, 0, i)),
        out_shape=jax.ShapeDtypeStruct((B, N, N), jnp.float32),
    )(xt, xt)
    return jnp.swapaxes(out_t, 1, 2)


# speedup vs baseline: 1.6312x; 1.1301x over previous
import functools

import jax
import jax.numpy as jnp
from jax.experimental import pallas as pl

_TOP_K = 32


def _xt_kernel(x_ref, w_ref, out_ref):
    xt = jnp.dot(x_ref[0], w_ref[...], preferred_element_type=jnp.float32)
    norm = jnp.sqrt(jnp.sum(xt * xt, axis=1, keepdims=True))
    out_ref[0] = xt / jnp.maximum(norm, 1e-12)


def _colsum_i16(x):
    n = x.shape[0]
    while n > 16:
        n //= 2
        x = x[:n] + x[n:]
    return jnp.sum(x.astype(jnp.int32), axis=0, keepdims=True)


def _scores_kernel(xt_all_ref, xt_blk_ref, out_ref):
    xa = xt_all_ref[0]
    xb = xt_blk_ref[0]
    st = jax.lax.dot_general(
        xa, xb, (((1,), (1,)), ((), ())),
        preferred_element_type=jnp.float32)
    st = jnp.maximum(st, 0.0)

    bm = st.shape[1]
    bits = jax.lax.bitcast_convert_type(st, jnp.int32)
    top16 = (bits >> 16).astype(jnp.int16)
    low16 = ((bits & 0xFFFF) - 32768).astype(jnp.int16)

    lo_t = jnp.zeros((1, bm), jnp.int32)
    hi_t = jnp.full((1, bm), 0x4000, jnp.int32)

    def body_top(_, lohi):
        lo, hi = lohi
        mid = (lo + hi) >> 1
        t = mid.astype(jnp.int16)
        cnt = _colsum_i16((top16 >= t).astype(jnp.int16))
        ge = cnt >= _TOP_K
        return jnp.where(ge, mid, lo), jnp.where(ge, hi, mid)

    lo_t, hi_t = jax.lax.fori_loop(0, 14, body_top, (lo_t, hi_t))

    t_top = lo_t.astype(jnp.int16)
    in_bucket = top16 == t_top
    cnt_above = _colsum_i16((top16 > t_top).astype(jnp.int16))
    need = _TOP_K - cnt_above

    lo_l = jnp.full((1, bm), -32768, jnp.int32)
    hi_l = jnp.full((1, bm), 32768, jnp.int32)

    def body_low(_, lohi):
        lo, hi = lohi
        mid = (lo + hi) >> 1
        t = mid.astype(jnp.int16)
        cnt = _colsum_i16((in_bucket & (low16 >= t)).astype(jnp.int16))
        ge = cnt >= need
        return jnp.where(ge, mid, lo), jnp.where(ge, hi, mid)

    lo_l, hi_l = jax.lax.fori_loop(0, 16, body_low, (lo_l, hi_l))

    thresh_bits = (lo_t << 16) | (lo_l + 32768)
    thresh = jax.lax.bitcast_convert_type(thresh_bits, jnp.float32)

    e = jnp.where(st >= thresh, jnp.exp(st), 1.0)
    res = e / jnp.sum(e, axis=0, keepdims=True)
    out_ref[0] = res.T


@functools.partial(jax.jit, static_argnames=())
def kernel(x, W):
    B, N, D = x.shape
    bm1 = 256
    xt = pl.pallas_call(
        _xt_kernel,
        grid=(B, N // bm1),
        in_specs=[
            pl.BlockSpec((1, bm1, D), lambda b, i: (b, i, 0)),
            pl.BlockSpec((D, D), lambda b, i: (0, 0)),
        ],
        out_specs=pl.BlockSpec((1, bm1, D), lambda b, i: (b, i, 0)),
        out_shape=jax.ShapeDtypeStruct((B, N, D), jnp.float32),
    )(x, W)

    bm2 = 256
    out_t = pl.pallas_call(
        _scores_kernel,
        grid=(B, N // bm2),
        in_specs=[
            pl.BlockSpec((1, N, D), lambda b, i: (b, 0, 0)),
            pl.BlockSpec((1, bm2, D), lambda b, i: (b, i, 0)),
        ],
        out_specs=pl.BlockSpec((1, bm2, N), lambda b, i: (b, i, 0)),
        out_shape=jax.ShapeDtypeStruct((B, N, N), jnp.float32),
    )(xt, xt)
    return out_t
